# Initial kernel scaffold; baseline (speedup 1.0000x reference)
import functools

import jax
import jax.numpy as jnp
from jax import lax
from jax.experimental import pallas as pl
from jax.experimental.pallas import tpu as pltpu
from jax.experimental.pallas import tpu_sc as plsc

VOCAB = 100000
EMBED = 256
BATCH = 4096
HIST = 200
LPAD = 256
NW = 32
ROWS_PER_W = BATCH // NW
DMA_W = 128
CH = ROWS_PER_W * LPAD // DMA_W
ROWS_PER_L = LPAD // DMA_W
NCHUNK = LPAD // 16
BLK = 2000


def _rowmean_body(pt_ref, ct_ref, pm_ref, cm_ref):
    pm_ref[...] = jnp.mean(pt_ref[...], axis=1, keepdims=True)
    cm_ref[...] = jnp.mean(ct_ref[...], axis=1, keepdims=True)


def _rowmeans(paragraph_table, context_table):
    spec_in = pl.BlockSpec((BLK, EMBED), lambda i: (i, 0))
    spec_out = pl.BlockSpec((BLK, 1), lambda i: (i, 0))
    pm, cm = pl.pallas_call(
        _rowmean_body,
        grid=(VOCAB // BLK,),
        in_specs=[spec_in, spec_in],
        out_specs=[spec_out, spec_out],
        out_shape=[jax.ShapeDtypeStruct((VOCAB, 1), jnp.float32)] * 2,
    )(paragraph_table, context_table)
    return pm[:, 0], cm[:, 0]


def _sc_gather_softmax(pm_ext, cm_ext, pidx_r, cidx_r):
    mesh = plsc.VectorSubcoreMesh(core_axis_name="c", subcore_axis_name="s")

    @functools.partial(
        pl.kernel,
        out_type=jax.ShapeDtypeStruct((BATCH * LPAD // DMA_W, DMA_W),
                                      jnp.float32),
        mesh=mesh,
        scratch_types=[
            pltpu.VMEM((CH, DMA_W), jnp.int32),
            pltpu.VMEM((CH, DMA_W), jnp.float32),
            pltpu.VMEM((CH, DMA_W), jnp.float32),
            pltpu.SemaphoreType.DMA,
        ],
    )
    def k(pm_hbm, cm_hbm, pidx_hbm, cidx_hbm, out_hbm, idx_v, pv, cv, sem):
        nc = lax.axis_size("c")
        wid = lax.axis_index("s") * nc + lax.axis_index("c")
        base = wid * CH
        pltpu.sync_copy(pidx_hbm.at[pl.ds(base, CH)], idx_v)
        pltpu.async_copy(pm_hbm.at[idx_v], pv, sem).wait()
        pltpu.sync_copy(cidx_hbm.at[pl.ds(base, CH)], idx_v)
        pltpu.async_copy(cm_hbm.at[idx_v], cv, sem).wait()

        def row(b, carry):
            r0 = ROWS_PER_L * b
            vm = jnp.full((16,), -1e30, jnp.float32)
            for j in range(NCHUNK):
                r = r0 + j // 8
                sl = pl.ds(16 * (j % 8), 16)
                v = pv[r, sl] + cv[r, sl]
                pv[r, sl] = v
                vm = jnp.maximum(vm, v)
            m = jnp.max(vm)
            vs = jnp.zeros((16,), jnp.float32)
            for j in range(NCHUNK):
                r = r0 + j // 8
                sl = pl.ds(16 * (j % 8), 16)
                e = jnp.exp(pv[r, sl] - m)
                pv[r, sl] = e
                vs = vs + e
            inv = 1.0 / jnp.sum(vs)
            for j in range(NCHUNK):
                r = r0 + j // 8
                sl = pl.ds(16 * (j % 8), 16)
                pv[r, sl] = pv[r, sl] * inv
            return carry

        lax.fori_loop(0, ROWS_PER_W, row, 0)
        pltpu.sync_copy(pv, out_hbm.at[pl.ds(base, CH)])

    return k(pm_ext, cm_ext, pidx_r, cidx_r)


def kernel(inputs, paragraph_table, context_table):
    pm, cm = _rowmeans(paragraph_table, context_table)
    sentinel = jnp.full((8,), -1e30, jnp.float32)
    pm_ext = jnp.concatenate([pm, sentinel])
    cm_ext = jnp.concatenate([cm, jnp.zeros((8,), jnp.float32)])

    pad = jnp.full((BATCH, LPAD - HIST), VOCAB, jnp.int32)
    pidx = jnp.concatenate([inputs[:, 0].astype(jnp.int32), pad], axis=1)
    cidx = jnp.concatenate([inputs[:, 1].astype(jnp.int32), pad], axis=1)
    pidx_r = pidx.reshape(-1, DMA_W)
    cidx_r = cidx.reshape(-1, DMA_W)

    out = _sc_gather_softmax(pm_ext, cm_ext, pidx_r, cidx_r)
    out = out.reshape(BATCH, LPAD)[:, :HIST]
    return out[:, None, :]


# speedup vs baseline: 12.4003x; 12.4003x over previous
import functools

import jax
import jax.numpy as jnp
from jax import lax
from jax.experimental import pallas as pl
from jax.experimental.pallas import tpu as pltpu
from jax.experimental.pallas import tpu_sc as plsc

VOCAB = 100000
EMBED = 256
BATCH = 4096
HIST = 200
NW = 32
ROWS_PER_W = BATCH // NW
G = ROWS_PER_W // 16
BLK = 2000


def _rowmean_body(pt_ref, ct_ref, pm_ref, cm_ref):
    pm_ref[...] = jnp.mean(pt_ref[...], axis=1, keepdims=True)
    cm_ref[...] = jnp.mean(ct_ref[...], axis=1, keepdims=True)


def _rowmeans(paragraph_table, context_table):
    spec_in = pl.BlockSpec((BLK, EMBED), lambda i: (i, 0))
    spec_out = pl.BlockSpec((BLK, 1), lambda i: (i, 0))
    pm, cm = pl.pallas_call(
        _rowmean_body,
        grid=(VOCAB // BLK,),
        in_specs=[spec_in, spec_in],
        out_specs=[spec_out, spec_out],
        out_shape=[jax.ShapeDtypeStruct((VOCAB, 1), jnp.float32)] * 2,
    )(paragraph_table, context_table)
    return pm[:, 0], cm[:, 0]


def _sc_gather_softmax(pm, cm, pidx_t, cidx_t):
    mesh = plsc.VectorSubcoreMesh(core_axis_name="c", subcore_axis_name="s")

    @functools.partial(
        pl.kernel,
        out_type=jax.ShapeDtypeStruct((NW * HIST, ROWS_PER_W), jnp.float32),
        mesh=mesh,
        scratch_types=[
            pltpu.VMEM((HIST, ROWS_PER_W), jnp.int32),
            pltpu.VMEM((HIST, ROWS_PER_W), jnp.int32),
            pltpu.VMEM((HIST, ROWS_PER_W), jnp.float32),
            pltpu.VMEM((HIST, ROWS_PER_W), jnp.float32),
            pltpu.SemaphoreType.DMA,
        ],
    )
    def k(pm_hbm, cm_hbm, pidx_hbm, cidx_hbm, out_hbm,
          pidx_v, cidx_v, pv, cv, sem):
        nc = lax.axis_size("c")
        wid = lax.axis_index("s") * nc + lax.axis_index("c")
        base = wid * HIST
        pltpu.sync_copy(pidx_hbm.at[pl.ds(base, HIST)], pidx_v)
        pltpu.sync_copy(cidx_hbm.at[pl.ds(base, HIST)], cidx_v)

        def fire(l, c):
            pltpu.make_async_copy(
                pm_hbm.at[pidx_v.at[l]], pv.at[l], sem).start()
            pltpu.make_async_copy(
                cm_hbm.at[cidx_v.at[l]], cv.at[l], sem).start()
            return c

        lax.fori_loop(0, HIST, fire, 0)
        pltpu.make_async_copy(out_hbm.at[pl.ds(base, HIST)], pv, sem).wait()
        pltpu.make_async_copy(out_hbm.at[pl.ds(base, HIST)], cv, sem).wait()

        def pass_max(l, vm):
            new = []
            for g in range(G):
                sl = pl.ds(16 * g, 16)
                v = pv[l, sl] + cv[l, sl]
                pv[l, sl] = v
                new.append(jnp.maximum(vm[g], v))
            return tuple(new)

        vm = lax.fori_loop(
            0, HIST, pass_max,
            tuple(jnp.full((16,), -1e30, jnp.float32) for _ in range(G)))

        def pass_exp(l, vs):
            new = []
            for g in range(G):
                sl = pl.ds(16 * g, 16)
                e = jnp.exp(pv[l, sl] - vm[g])
                pv[l, sl] = e
                new.append(vs[g] + e)
            return tuple(new)

        vs = lax.fori_loop(
            0, HIST, pass_exp,
            tuple(jnp.zeros((16,), jnp.float32) for _ in range(G)))

        inv = tuple(1.0 / vs[g] for g in range(G))

        def pass_norm(l, c):
            for g in range(G):
                sl = pl.ds(16 * g, 16)
                pv[l, sl] = pv[l, sl] * inv[g]
            return c

        lax.fori_loop(0, HIST, pass_norm, 0)
        pltpu.sync_copy(pv, out_hbm.at[pl.ds(base, HIST)])

    return k(pm, cm, pidx_t, cidx_t)


def _to_lb(idx2d):
    return (idx2d.reshape(NW, ROWS_PER_W, HIST)
            .transpose(0, 2, 1)
            .reshape(NW * HIST, ROWS_PER_W))


def kernel(inputs, paragraph_table, context_table):
    pm, cm = _rowmeans(paragraph_table, context_table)
    pidx_t = _to_lb(inputs[:, 0].astype(jnp.int32))
    cidx_t = _to_lb(inputs[:, 1].astype(jnp.int32))
    out_t = _sc_gather_softmax(pm, cm, pidx_t, cidx_t)
    out = (out_t.reshape(NW, HIST, ROWS_PER_W)
           .transpose(0, 2, 1)
           .reshape(BATCH, HIST))
    return out[:, None, :]


# speedup vs baseline: 17.1116x; 1.3799x over previous
import functools

import jax
import jax.numpy as jnp
from jax import lax
from jax.experimental import pallas as pl
from jax.experimental.pallas import tpu as pltpu
from jax.experimental.pallas import tpu_sc as plsc

VOCAB = 100000
EMBED = 256
BATCH = 4096
HIST = 200
NW = 32
ROWS_PER_W = BATCH // NW
G = ROWS_PER_W // 16
BLK = 2000
CHROWS = 8
NCHUNK = HIST // CHROWS


def _rowmean_pack_body(pt_ref, ct_ref, tab_ref):
    pm = jnp.mean(pt_ref[...], axis=1, keepdims=True)
    cm = jnp.mean(ct_ref[...], axis=1, keepdims=True)
    pm16 = lax.bitcast_convert_type(pm.astype(jnp.bfloat16), jnp.uint16)
    cm16 = lax.bitcast_convert_type(cm.astype(jnp.bfloat16), jnp.uint16)
    word = (pm16.astype(jnp.uint32) << 16) | cm16.astype(jnp.uint32)
    tab_ref[...] = lax.bitcast_convert_type(word, jnp.int32)


def _packed_rowmeans(paragraph_table, context_table):
    spec_in = pl.BlockSpec((BLK, EMBED), lambda i: (i, 0))
    spec_out = pl.BlockSpec((BLK, 1), lambda i: (i, 0))
    tab = pl.pallas_call(
        _rowmean_pack_body,
        grid=(VOCAB // BLK,),
        in_specs=[spec_in, spec_in],
        out_specs=spec_out,
        out_shape=jax.ShapeDtypeStruct((VOCAB, 1), jnp.int32),
    )(paragraph_table, context_table)
    return tab[:, 0]


def _sc_gather_softmax(tab, pidx_t, cidx_t):
    mesh = plsc.VectorSubcoreMesh(core_axis_name="c", subcore_axis_name="s")
    chunk_bytes = CHROWS * ROWS_PER_W * 4
    hi_mask = jnp.int32(-65536)

    @functools.partial(
        pl.kernel,
        out_type=jax.ShapeDtypeStruct((NW * HIST, ROWS_PER_W), jnp.float32),
        mesh=mesh,
        scratch_types=[
            pltpu.VMEM((VOCAB,), jnp.int32),
            pltpu.VMEM((2, CHROWS, ROWS_PER_W), jnp.int32),
            pltpu.VMEM((2, CHROWS, ROWS_PER_W), jnp.int32),
            pltpu.VMEM((HIST, ROWS_PER_W), jnp.float32),
            pltpu.SemaphoreType.DMA,
            pltpu.SemaphoreType.DMA,
            pltpu.SemaphoreType.DMA,
        ],
        compiler_params=pltpu.CompilerParams(needs_layout_passes=False),
    )
    def k(tab_hbm, pidx_hbm, cidx_hbm, out_hbm,
          tab_v, pidx_c, cidx_c, vv, sem_t, sem_p, sem_c):
        nc = lax.axis_size("c")
        wid = lax.axis_index("s") * nc + lax.axis_index("c")
        base = wid * HIST

        pltpu.make_async_copy(tab_hbm, tab_v, sem_t).start()

        def issue(ci, bd):
            pltpu.make_async_copy(
                pidx_hbm.at[pl.ds(base + ci * CHROWS, CHROWS)],
                pidx_c.at[bd], sem_p).start()
            pltpu.make_async_copy(
                cidx_hbm.at[pl.ds(base + ci * CHROWS, CHROWS)],
                cidx_c.at[bd], sem_c).start()

        issue(0, 0)
        pltpu.make_async_copy(tab_hbm, tab_v, sem_t).wait()

        def chunk(ci, vm):
            bd = lax.rem(ci, 2)

            @pl.when(ci + 1 < NCHUNK)
            def _():
                issue(ci + 1, lax.rem(ci + 1, 2))

            pltpu.make_async_copy(
                pidx_hbm.at[pl.ds(base, CHROWS)], pidx_c.at[0], sem_p).wait()
            pltpu.make_async_copy(
                cidx_hbm.at[pl.ds(base, CHROWS)], cidx_c.at[0], sem_c).wait()

            l0 = ci * CHROWS
            new_vm = list(vm)
            for r in range(CHROWS):
                for g in range(G):
                    sl = pl.ds(16 * g, 16)
                    wp = plsc.load_gather(tab_v, [pidx_c[bd, r, sl]])
                    wc = plsc.load_gather(tab_v, [cidx_c[bd, r, sl]])
                    vp = plsc.bitcast(wp & hi_mask, jnp.float32)
                    vc = plsc.bitcast(wc << 16, jnp.float32)
                    v = vp + vc
                    vv[l0 + r, sl] = v
                    new_vm[g] = jnp.maximum(new_vm[g], v)
            return tuple(new_vm)

        vm = lax.fori_loop(
            0, NCHUNK, chunk,
            tuple(jnp.full((16,), -1e30, jnp.float32) for _ in range(G)))

        def pass_exp(l, vs):
            new = []
            for g in range(G):
                sl = pl.ds(16 * g, 16)
                e = jnp.exp(vv[l, sl] - vm[g])
                vv[l, sl] = e
                new.append(vs[g] + e)
            return tuple(new)

        vs = lax.fori_loop(
            0, HIST, pass_exp,
            tuple(jnp.zeros((16,), jnp.float32) for _ in range(G)))

        inv = tuple(1.0 / vs[g] for g in range(G))

        def pass_norm(l, c):
            for g in range(G):
                sl = pl.ds(16 * g, 16)
                vv[l, sl] = vv[l, sl] * inv[g]
            return c

        lax.fori_loop(0, HIST, pass_norm, 0)
        pltpu.sync_copy(vv, out_hbm.at[pl.ds(base, HIST)])

    del chunk_bytes
    return k(tab, pidx_t, cidx_t)


def _to_lb(idx2d):
    return (idx2d.reshape(NW, ROWS_PER_W, HIST)
            .transpose(0, 2, 1)
            .reshape(NW * HIST, ROWS_PER_W))


def kernel(inputs, paragraph_table, context_table):
    tab = _packed_rowmeans(paragraph_table, context_table)
    pidx_t = _to_lb(inputs[:, 0].astype(jnp.int32))
    cidx_t = _to_lb(inputs[:, 1].astype(jnp.int32))
    out_t = _sc_gather_softmax(tab, pidx_t, cidx_t)
    out = (out_t.reshape(NW, HIST, ROWS_PER_W)
           .transpose(0, 2, 1)
           .reshape(BATCH, HIST))
    return out[:, None, :]


# speedup vs baseline: 17.9186x; 1.0472x over previous
import functools

import jax
import jax.numpy as jnp
from jax import lax
from jax.experimental import pallas as pl
from jax.experimental.pallas import tpu as pltpu
from jax.experimental.pallas import tpu_sc as plsc

VOCAB = 100000
EMBED = 256
BATCH = 4096
HIST = 200
NW = 32
ROWS_PER_W = BATCH // NW
G = ROWS_PER_W // 16
BLK = 4000
CHROWS = 8
NCHUNK = HIST // CHROWS


def _rowmean_pack_body(pt_ref, ct_ref, tab_ref):
    pm = jnp.mean(pt_ref[...], axis=1, keepdims=True)
    cm = jnp.mean(ct_ref[...], axis=1, keepdims=True)
    pm16 = lax.bitcast_convert_type(pm.astype(jnp.bfloat16), jnp.uint16)
    cm16 = lax.bitcast_convert_type(cm.astype(jnp.bfloat16), jnp.uint16)
    word = (pm16.astype(jnp.uint32) << 16) | cm16.astype(jnp.uint32)
    tab_ref[...] = lax.bitcast_convert_type(word, jnp.int32)


def _packed_rowmeans(paragraph_table, context_table):
    spec_in = pl.BlockSpec((BLK, EMBED), lambda i: (i, 0))
    spec_out = pl.BlockSpec((BLK, 1), lambda i: (i, 0))
    tab = pl.pallas_call(
        _rowmean_pack_body,
        grid=(VOCAB // BLK,),
        in_specs=[spec_in, spec_in],
        out_specs=spec_out,
        out_shape=jax.ShapeDtypeStruct((VOCAB, 1), jnp.int32),
    )(paragraph_table, context_table)
    return tab[:, 0]


def _sc_gather_softmax(tab, pidx_t, cidx_t):
    mesh = plsc.VectorSubcoreMesh(core_axis_name="c", subcore_axis_name="s")
    chunk_bytes = CHROWS * ROWS_PER_W * 4
    hi_mask = jnp.int32(-65536)

    @functools.partial(
        pl.kernel,
        out_type=jax.ShapeDtypeStruct((NW * HIST, ROWS_PER_W), jnp.float32),
        mesh=mesh,
        scratch_types=[
            pltpu.VMEM((VOCAB,), jnp.int32),
            pltpu.VMEM((2, CHROWS, ROWS_PER_W), jnp.int32),
            pltpu.VMEM((2, CHROWS, ROWS_PER_W), jnp.int32),
            pltpu.VMEM((HIST, ROWS_PER_W), jnp.float32),
            pltpu.SemaphoreType.DMA,
            pltpu.SemaphoreType.DMA,
            pltpu.SemaphoreType.DMA,
        ],
        compiler_params=pltpu.CompilerParams(needs_layout_passes=False),
    )
    def k(tab_hbm, pidx_hbm, cidx_hbm, out_hbm,
          tab_v, pidx_c, cidx_c, vv, sem_t, sem_p, sem_c):
        nc = lax.axis_size("c")
        wid = lax.axis_index("s") * nc + lax.axis_index("c")
        base = wid * HIST

        pltpu.make_async_copy(tab_hbm, tab_v, sem_t).start()

        def issue(ci, bd):
            pltpu.make_async_copy(
                pidx_hbm.at[pl.ds(base + ci * CHROWS, CHROWS)],
                pidx_c.at[bd], sem_p).start()
            pltpu.make_async_copy(
                cidx_hbm.at[pl.ds(base + ci * CHROWS, CHROWS)],
                cidx_c.at[bd], sem_c).start()

        issue(0, 0)
        pltpu.make_async_copy(tab_hbm, tab_v, sem_t).wait()

        def chunk(ci, vm):
            bd = lax.rem(ci, 2)

            @pl.when(ci + 1 < NCHUNK)
            def _():
                issue(ci + 1, lax.rem(ci + 1, 2))

            pltpu.make_async_copy(
                pidx_hbm.at[pl.ds(base, CHROWS)], pidx_c.at[0], sem_p).wait()
            pltpu.make_async_copy(
                cidx_hbm.at[pl.ds(base, CHROWS)], cidx_c.at[0], sem_c).wait()

            l0 = ci * CHROWS
            new_vm = list(vm)
            for r in range(CHROWS):
                for g in range(G):
                    sl = pl.ds(16 * g, 16)
                    wp = plsc.load_gather(tab_v, [pidx_c[bd, r, sl]])
                    wc = plsc.load_gather(tab_v, [cidx_c[bd, r, sl]])
                    vp = plsc.bitcast(wp & hi_mask, jnp.float32)
                    vc = plsc.bitcast(wc << 16, jnp.float32)
                    v = vp + vc
                    vv[l0 + r, sl] = v
                    new_vm[g] = jnp.maximum(new_vm[g], v)
            return tuple(new_vm)

        vm = lax.fori_loop(
            0, NCHUNK, chunk,
            tuple(jnp.full((16,), -1e30, jnp.float32) for _ in range(G)))

        def pass_exp(l, vs):
            new = []
            for g in range(G):
                sl = pl.ds(16 * g, 16)
                e = jnp.exp(vv[l, sl] - vm[g])
                vv[l, sl] = e
                new.append(vs[g] + e)
            return tuple(new)

        vs = lax.fori_loop(
            0, HIST, pass_exp,
            tuple(jnp.zeros((16,), jnp.float32) for _ in range(G)))

        inv = tuple(1.0 / vs[g] for g in range(G))

        def pass_norm(l, c):
            for g in range(G):
                sl = pl.ds(16 * g, 16)
                vv[l, sl] = vv[l, sl] * inv[g]
            return c

        lax.fori_loop(0, HIST, pass_norm, 0)
        pltpu.sync_copy(vv, out_hbm.at[pl.ds(base, HIST)])

    del chunk_bytes
    return k(tab, pidx_t, cidx_t)


def _to_lb(idx2d):
    return (idx2d.reshape(NW, ROWS_PER_W, HIST)
            .transpose(0, 2, 1)
            .reshape(NW * HIST, ROWS_PER_W))


def kernel(inputs, paragraph_table, context_table):
    tab = _packed_rowmeans(paragraph_table, context_table)
    pidx_t = _to_lb(inputs[:, 0].astype(jnp.int32))
    cidx_t = _to_lb(inputs[:, 1].astype(jnp.int32))
    out_t = _sc_gather_softmax(tab, pidx_t, cidx_t)
    out = (out_t.reshape(NW, HIST, ROWS_PER_W)
           .transpose(0, 2, 1)
           .reshape(BATCH, HIST))
    return out[:, None, :]


# speedup vs baseline: 17.9217x; 1.0002x over previous
import functools

import jax
import jax.numpy as jnp
from jax import lax
from jax.experimental import pallas as pl
from jax.experimental.pallas import tpu as pltpu
from jax.experimental.pallas import tpu_sc as plsc

VOCAB = 100000
EMBED = 256
BATCH = 4096
HIST = 200
NW = 32
ROWS_PER_W = BATCH // NW
G = ROWS_PER_W // 16
BLK = 5000
CHROWS = 8
NCHUNK = HIST // CHROWS


def _rowmean_pack_body(pt_ref, ct_ref, tab_ref):
    pm = jnp.mean(pt_ref[...], axis=1, keepdims=True)
    cm = jnp.mean(ct_ref[...], axis=1, keepdims=True)
    pm16 = lax.bitcast_convert_type(pm.astype(jnp.bfloat16), jnp.uint16)
    cm16 = lax.bitcast_convert_type(cm.astype(jnp.bfloat16), jnp.uint16)
    word = (pm16.astype(jnp.uint32) << 16) | cm16.astype(jnp.uint32)
    tab_ref[...] = lax.bitcast_convert_type(word, jnp.int32)


def _packed_rowmeans(paragraph_table, context_table):
    spec_in = pl.BlockSpec((BLK, EMBED), lambda i: (i, 0))
    spec_out = pl.BlockSpec((BLK, 1), lambda i: (i, 0))
    tab = pl.pallas_call(
        _rowmean_pack_body,
        grid=(VOCAB // BLK,),
        in_specs=[spec_in, spec_in],
        out_specs=spec_out,
        out_shape=jax.ShapeDtypeStruct((VOCAB, 1), jnp.int32),
    )(paragraph_table, context_table)
    return tab[:, 0]


def _sc_gather_softmax(tab, pidx_t, cidx_t):
    mesh = plsc.VectorSubcoreMesh(core_axis_name="c", subcore_axis_name="s")
    chunk_bytes = CHROWS * ROWS_PER_W * 4
    hi_mask = jnp.int32(-65536)

    @functools.partial(
        pl.kernel,
        out_type=jax.ShapeDtypeStruct((NW * HIST, ROWS_PER_W), jnp.float32),
        mesh=mesh,
        scratch_types=[
            pltpu.VMEM((VOCAB,), jnp.int32),
            pltpu.VMEM((2, CHROWS, ROWS_PER_W), jnp.int32),
            pltpu.VMEM((2, CHROWS, ROWS_PER_W), jnp.int32),
            pltpu.VMEM((HIST, ROWS_PER_W), jnp.float32),
            pltpu.SemaphoreType.DMA,
            pltpu.SemaphoreType.DMA,
            pltpu.SemaphoreType.DMA,
        ],
        compiler_params=pltpu.CompilerParams(needs_layout_passes=False),
    )
    def k(tab_hbm, pidx_hbm, cidx_hbm, out_hbm,
          tab_v, pidx_c, cidx_c, vv, sem_t, sem_p, sem_c):
        nc = lax.axis_size("c")
        wid = lax.axis_index("s") * nc + lax.axis_index("c")
        base = wid * HIST

        pltpu.make_async_copy(tab_hbm, tab_v, sem_t).start()

        def issue(ci, bd):
            pltpu.make_async_copy(
                pidx_hbm.at[pl.ds(base + ci * CHROWS, CHROWS)],
                pidx_c.at[bd], sem_p).start()
            pltpu.make_async_copy(
                cidx_hbm.at[pl.ds(base + ci * CHROWS, CHROWS)],
                cidx_c.at[bd], sem_c).start()

        issue(0, 0)
        pltpu.make_async_copy(tab_hbm, tab_v, sem_t).wait()

        def chunk(ci, vm):
            bd = lax.rem(ci, 2)

            @pl.when(ci + 1 < NCHUNK)
            def _():
                issue(ci + 1, lax.rem(ci + 1, 2))

            pltpu.make_async_copy(
                pidx_hbm.at[pl.ds(base, CHROWS)], pidx_c.at[0], sem_p).wait()
            pltpu.make_async_copy(
                cidx_hbm.at[pl.ds(base, CHROWS)], cidx_c.at[0], sem_c).wait()

            l0 = ci * CHROWS
            new_vm = list(vm)
            for r in range(CHROWS):
                for g in range(G):
                    sl = pl.ds(16 * g, 16)
                    wp = plsc.load_gather(tab_v, [pidx_c[bd, r, sl]])
                    wc = plsc.load_gather(tab_v, [cidx_c[bd, r, sl]])
                    vp = plsc.bitcast(wp & hi_mask, jnp.float32)
                    vc = plsc.bitcast(wc << 16, jnp.float32)
                    v = vp + vc
                    vv[l0 + r, sl] = v
                    new_vm[g] = jnp.maximum(new_vm[g], v)
            return tuple(new_vm)

        vm = lax.fori_loop(
            0, NCHUNK, chunk,
            tuple(jnp.full((16,), -1e30, jnp.float32) for _ in range(G)))

        def pass_exp(l, vs):
            new = []
            for g in range(G):
                sl = pl.ds(16 * g, 16)
                e = jnp.exp(vv[l, sl] - vm[g])
                vv[l, sl] = e
                new.append(vs[g] + e)
            return tuple(new)

        vs = lax.fori_loop(
            0, HIST, pass_exp,
            tuple(jnp.zeros((16,), jnp.float32) for _ in range(G)))

        inv = tuple(1.0 / vs[g] for g in range(G))

        def pass_norm(l, c):
            for g in range(G):
                sl = pl.ds(16 * g, 16)
                vv[l, sl] = vv[l, sl] * inv[g]
            return c

        lax.fori_loop(0, HIST, pass_norm, 0)
        pltpu.sync_copy(vv, out_hbm.at[pl.ds(base, HIST)])

    del chunk_bytes
    return k(tab, pidx_t, cidx_t)


def _to_lb(idx2d):
    return (idx2d.reshape(NW, ROWS_PER_W, HIST)
            .transpose(0, 2, 1)
            .reshape(NW * HIST, ROWS_PER_W))


def kernel(inputs, paragraph_table, context_table):
    tab = _packed_rowmeans(paragraph_table, context_table)
    pidx_t = _to_lb(inputs[:, 0].astype(jnp.int32))
    cidx_t = _to_lb(inputs[:, 1].astype(jnp.int32))
    out_t = _sc_gather_softmax(tab, pidx_t, cidx_t)
    out = (out_t.reshape(NW, HIST, ROWS_PER_W)
           .transpose(0, 2, 1)
           .reshape(BATCH, HIST))
    return out[:, None, :]


# speedup vs baseline: 17.9369x; 1.0008x over previous
import functools

import jax
import jax.numpy as jnp
from jax import lax
from jax.experimental import pallas as pl
from jax.experimental.pallas import tpu as pltpu
from jax.experimental.pallas import tpu_sc as plsc

VOCAB = 100000
EMBED = 256
BATCH = 4096
HIST = 200
NW = 32
ROWS_PER_W = BATCH // NW
G = ROWS_PER_W // 16
BLK = 5000
CHROWS = 8
NCHUNK = HIST // CHROWS


def _rowmean_pack_body(pt_ref, ct_ref, tab_ref):
    pm = jnp.mean(pt_ref[...], axis=1, keepdims=True)
    cm = jnp.mean(ct_ref[...], axis=1, keepdims=True)
    pm16 = lax.bitcast_convert_type(pm.astype(jnp.bfloat16), jnp.uint16)
    cm16 = lax.bitcast_convert_type(cm.astype(jnp.bfloat16), jnp.uint16)
    word = (pm16.astype(jnp.uint32) << 16) | cm16.astype(jnp.uint32)
    tab_ref[...] = lax.bitcast_convert_type(word, jnp.int32)


def _packed_rowmeans(paragraph_table, context_table):
    spec_in = pl.BlockSpec((BLK, EMBED), lambda i: (i, 0))
    spec_out = pl.BlockSpec((BLK, 1), lambda i: (i, 0))
    tab = pl.pallas_call(
        _rowmean_pack_body,
        grid=(VOCAB // BLK,),
        in_specs=[spec_in, spec_in],
        out_specs=spec_out,
        out_shape=jax.ShapeDtypeStruct((VOCAB, 1), jnp.int32),
    )(paragraph_table, context_table)
    return tab[:, 0]


def _sc_gather_softmax(tab, pidx_t, cidx_t):
    mesh = plsc.VectorSubcoreMesh(core_axis_name="c", subcore_axis_name="s")
    chunk_bytes = CHROWS * ROWS_PER_W * 4
    hi_mask = jnp.int32(-65536)

    @functools.partial(
        pl.kernel,
        out_type=jax.ShapeDtypeStruct((NW * HIST, ROWS_PER_W), jnp.float32),
        mesh=mesh,
        scratch_types=[
            pltpu.VMEM((VOCAB,), jnp.int32),
            pltpu.VMEM((2, CHROWS, ROWS_PER_W), jnp.int32),
            pltpu.VMEM((2, CHROWS, ROWS_PER_W), jnp.int32),
            pltpu.VMEM((HIST, ROWS_PER_W), jnp.float32),
            pltpu.SemaphoreType.DMA,
            pltpu.SemaphoreType.DMA,
            pltpu.SemaphoreType.DMA,
        ],
        compiler_params=pltpu.CompilerParams(needs_layout_passes=False),
    )
    def k(tab_hbm, pidx_hbm, cidx_hbm, out_hbm,
          tab_v, pidx_c, cidx_c, vv, sem_t, sem_p, sem_c):
        nc = lax.axis_size("c")
        wid = lax.axis_index("s") * nc + lax.axis_index("c")
        base = wid * HIST

        pltpu.make_async_copy(tab_hbm, tab_v, sem_t).start()

        def issue(ci, bd):
            pltpu.make_async_copy(
                pidx_hbm.at[pl.ds(base + ci * CHROWS, CHROWS)],
                pidx_c.at[bd], sem_p).start()
            pltpu.make_async_copy(
                cidx_hbm.at[pl.ds(base + ci * CHROWS, CHROWS)],
                cidx_c.at[bd], sem_c).start()

        issue(0, 0)
        with jax.named_scope("tab_wait"):
            pltpu.make_async_copy(tab_hbm, tab_v, sem_t).wait()

        def chunk(ci, vm):
            bd = lax.rem(ci, 2)

            @pl.when(ci + 1 < NCHUNK)
            def _():
                issue(ci + 1, lax.rem(ci + 1, 2))

            pltpu.make_async_copy(
                pidx_hbm.at[pl.ds(base, CHROWS)], pidx_c.at[0], sem_p).wait()
            pltpu.make_async_copy(
                cidx_hbm.at[pl.ds(base, CHROWS)], cidx_c.at[0], sem_c).wait()

            l0 = ci * CHROWS
            new_vm = list(vm)
            for r in range(CHROWS):
                for g in range(G):
                    sl = pl.ds(16 * g, 16)
                    wp = plsc.load_gather(tab_v, [pidx_c[bd, r, sl]])
                    wc = plsc.load_gather(tab_v, [cidx_c[bd, r, sl]])
                    vp = plsc.bitcast(wp & hi_mask, jnp.float32)
                    vc = plsc.bitcast(wc << 16, jnp.float32)
                    v = vp + vc
                    vv[l0 + r, sl] = v
                    new_vm[g] = jnp.maximum(new_vm[g], v)
            return tuple(new_vm)

        with jax.named_scope("gather"):
            vm = lax.fori_loop(
                0, NCHUNK, chunk,
                tuple(jnp.full((16,), -1e30, jnp.float32) for _ in range(G)))

        def pass_exp(l, vs):
            new = []
            for g in range(G):
                sl = pl.ds(16 * g, 16)
                e = jnp.exp(vv[l, sl] - vm[g])
                vv[l, sl] = e
                new.append(vs[g] + e)
            return tuple(new)

        with jax.named_scope("exp"):
            vs = lax.fori_loop(
                0, HIST, pass_exp,
                tuple(jnp.zeros((16,), jnp.float32) for _ in range(G)))

        inv = tuple(1.0 / vs[g] for g in range(G))

        def pass_norm(l, c):
            for g in range(G):
                sl = pl.ds(16 * g, 16)
                vv[l, sl] = vv[l, sl] * inv[g]
            return c

        with jax.named_scope("norm"):
            lax.fori_loop(0, HIST, pass_norm, 0)
        with jax.named_scope("out"):
            pltpu.sync_copy(vv, out_hbm.at[pl.ds(base, HIST)])

    del chunk_bytes
    return k(tab, pidx_t, cidx_t)


def _to_lb(idx2d):
    return (idx2d.reshape(NW, ROWS_PER_W, HIST)
            .transpose(0, 2, 1)
            .reshape(NW * HIST, ROWS_PER_W))


def kernel(inputs, paragraph_table, context_table):
    tab = _packed_rowmeans(paragraph_table, context_table)
    pidx_t = _to_lb(inputs[:, 0].astype(jnp.int32))
    cidx_t = _to_lb(inputs[:, 1].astype(jnp.int32))
    out_t = _sc_gather_softmax(tab, pidx_t, cidx_t)
    out = (out_t.reshape(NW, HIST, ROWS_PER_W)
           .transpose(0, 2, 1)
           .reshape(BATCH, HIST))
    return out[:, None, :]


# speedup vs baseline: 19.8108x; 1.1045x over previous
import functools

import jax
import jax.numpy as jnp
from jax import lax
from jax.experimental import pallas as pl
from jax.experimental.pallas import tpu as pltpu
from jax.experimental.pallas import tpu_sc as plsc

VOCAB = 100000
EMBED = 256
BATCH = 4096
HIST = 200
NW = 32
ROWS_PER_W = BATCH // NW
G = ROWS_PER_W // 16
BLK = 4096
CHROWS = 8
NCHUNK = HIST // CHROWS


def _rowmean_pack_body(pt_ref, ct_ref, tab_ref):
    pm = jnp.mean(pt_ref[...], axis=1, keepdims=True)
    cm = jnp.mean(ct_ref[...], axis=1, keepdims=True)
    pm16 = lax.bitcast_convert_type(pm.astype(jnp.bfloat16), jnp.uint16)
    cm16 = lax.bitcast_convert_type(cm.astype(jnp.bfloat16), jnp.uint16)
    word = (pm16.astype(jnp.uint32) << 16) | cm16.astype(jnp.uint32)
    tab_ref[...] = lax.bitcast_convert_type(word, jnp.int32)[:, 0]


def _packed_rowmeans(paragraph_table, context_table):
    spec_in = pl.BlockSpec((BLK, EMBED), lambda i: (i, 0))
    spec_out = pl.BlockSpec((BLK,), lambda i: (i,))
    tab = pl.pallas_call(
        _rowmean_pack_body,
        grid=(pl.cdiv(VOCAB, BLK),),
        in_specs=[spec_in, spec_in],
        out_specs=spec_out,
        out_shape=jax.ShapeDtypeStruct((VOCAB,), jnp.int32),
    )(paragraph_table, context_table)
    return tab


def _sc_gather_softmax(tab, pidx_t, cidx_t):
    mesh = plsc.VectorSubcoreMesh(core_axis_name="c", subcore_axis_name="s")
    chunk_bytes = CHROWS * ROWS_PER_W * 4
    hi_mask = jnp.int32(-65536)

    @functools.partial(
        pl.kernel,
        out_type=jax.ShapeDtypeStruct((NW * HIST, ROWS_PER_W), jnp.float32),
        mesh=mesh,
        scratch_types=[
            pltpu.VMEM((VOCAB,), jnp.int32),
            pltpu.VMEM((2, CHROWS, ROWS_PER_W), jnp.int32),
            pltpu.VMEM((2, CHROWS, ROWS_PER_W), jnp.int32),
            pltpu.VMEM((HIST, ROWS_PER_W), jnp.float32),
            pltpu.SemaphoreType.DMA,
            pltpu.SemaphoreType.DMA,
            pltpu.SemaphoreType.DMA,
        ],
        compiler_params=pltpu.CompilerParams(needs_layout_passes=False),
    )
    def k(tab_hbm, pidx_hbm, cidx_hbm, out_hbm,
          tab_v, pidx_c, cidx_c, vv, sem_t, sem_p, sem_c):
        nc = lax.axis_size("c")
        wid = lax.axis_index("s") * nc + lax.axis_index("c")
        base = wid * HIST

        pltpu.make_async_copy(tab_hbm, tab_v, sem_t).start()

        def issue(ci, bd):
            pltpu.make_async_copy(
                pidx_hbm.at[pl.ds(base + ci * CHROWS, CHROWS)],
                pidx_c.at[bd], sem_p).start()
            pltpu.make_async_copy(
                cidx_hbm.at[pl.ds(base + ci * CHROWS, CHROWS)],
                cidx_c.at[bd], sem_c).start()

        issue(0, 0)
        with jax.named_scope("tab_wait"):
            pltpu.make_async_copy(tab_hbm, tab_v, sem_t).wait()

        def chunk(ci, vm):
            bd = lax.rem(ci, 2)

            @pl.when(ci + 1 < NCHUNK)
            def _():
                issue(ci + 1, lax.rem(ci + 1, 2))

            pltpu.make_async_copy(
                pidx_hbm.at[pl.ds(base, CHROWS)], pidx_c.at[0], sem_p).wait()
            pltpu.make_async_copy(
                cidx_hbm.at[pl.ds(base, CHROWS)], cidx_c.at[0], sem_c).wait()

            l0 = ci * CHROWS
            new_vm = list(vm)
            for r in range(CHROWS):
                for g in range(G):
                    sl = pl.ds(16 * g, 16)
                    wp = plsc.load_gather(tab_v, [pidx_c[bd, r, sl]])
                    wc = plsc.load_gather(tab_v, [cidx_c[bd, r, sl]])
                    vp = plsc.bitcast(wp & hi_mask, jnp.float32)
                    vc = plsc.bitcast(wc << 16, jnp.float32)
                    v = vp + vc
                    vv[l0 + r, sl] = v
                    new_vm[g] = jnp.maximum(new_vm[g], v)
            return tuple(new_vm)

        with jax.named_scope("gather"):
            vm = lax.fori_loop(
                0, NCHUNK, chunk,
                tuple(jnp.full((16,), -1e30, jnp.float32) for _ in range(G)))

        def pass_exp(l, vs):
            new = []
            for g in range(G):
                sl = pl.ds(16 * g, 16)
                e = jnp.exp(vv[l, sl] - vm[g])
                vv[l, sl] = e
                new.append(vs[g] + e)
            return tuple(new)

        with jax.named_scope("exp"):
            vs = lax.fori_loop(
                0, HIST, pass_exp,
                tuple(jnp.zeros((16,), jnp.float32) for _ in range(G)))

        inv = tuple(1.0 / vs[g] for g in range(G))

        def pass_norm(l, c):
            for g in range(G):
                sl = pl.ds(16 * g, 16)
                vv[l, sl] = vv[l, sl] * inv[g]
            return c

        with jax.named_scope("norm"):
            lax.fori_loop(0, HIST, pass_norm, 0)
        with jax.named_scope("out"):
            pltpu.sync_copy(vv, out_hbm.at[pl.ds(base, HIST)])

    del chunk_bytes
    return k(tab, pidx_t, cidx_t)


def _to_lb(idx2d):
    return (idx2d.reshape(NW, ROWS_PER_W, HIST)
            .transpose(0, 2, 1)
            .reshape(NW * HIST, ROWS_PER_W))


def kernel(inputs, paragraph_table, context_table):
    tab = _packed_rowmeans(paragraph_table, context_table)
    pidx_t = _to_lb(inputs[:, 0].astype(jnp.int32))
    cidx_t = _to_lb(inputs[:, 1].astype(jnp.int32))
    out_t = _sc_gather_softmax(tab, pidx_t, cidx_t)
    out = (out_t.reshape(NW, HIST, ROWS_PER_W)
           .transpose(0, 2, 1)
           .reshape(BATCH, HIST))
    return out[:, None, :]
